# baseline (speedup 1.0000x reference)
import functools

import jax
import jax.numpy as jnp
from jax import lax
from jax.experimental import pallas as pl
from jax.experimental.pallas import tpu as pltpu
from jax.experimental.pallas import tpu_sc as plsc

N = 10000
E = 320000
D = 128
D_OUT = 64
N_CLU = 16

NC = 2
NS = 16
L = 16
NW = NC * NS
EPT = E // NW
K = 16
NCHUNK = EPT // K
NBUF = 10
NESLOT = NBUF + 2
NOUTER = NCHUNK // NBUF
TAIL = NCHUNK - NOUTER * NBUF

WB_TILES = 10
WB_ROWS = N // WB_TILES

BM = 2000
GRID = N // BM


def _spmm_body(tab, srcs, dsts, ws, zeros, out, srcring, dstring, wring,
               buf, accum, sem_e, sem_b):
  cid = lax.axis_index("c")
  sid = lax.axis_index("s")
  wid = sid * NC + cid

  @pl.when(sid < WB_TILES)
  def _():
    pltpu.sync_copy(zeros.at[pl.ds(sid * WB_ROWS, WB_ROWS)],
                    accum.at[pl.ds(sid * WB_ROWS, WB_ROWS)])
  plsc.subcore_barrier()

  def fire_edge(c, s):
    pltpu.async_copy(srcs.at[wid, c], srcring.at[s], sem_e.at[s])
    pltpu.async_copy(dsts.at[wid, c], dstring.at[s], sem_e.at[s])

  def wait_edge(s):
    pltpu.make_async_copy(srcs.at[wid, 0], srcring.at[s], sem_e.at[s]).wait()
    pltpu.make_async_copy(dsts.at[wid, 0], dstring.at[s], sem_e.at[s]).wait()

  def fire_gather(c, c_slot, b):
    pltpu.async_copy(tab.at[srcring.at[c_slot]], buf.at[b], sem_b.at[b])
    pltpu.async_copy(ws.at[wid, c], wring.at[b], sem_b.at[b])

  def wait_gather(b):
    pltpu.make_async_copy(tab.at[srcring.at[0]], buf.at[b],
                          sem_b.at[b]).wait()
    pltpu.make_async_copy(ws.at[wid, 0], wring.at[b], sem_b.at[b]).wait()

  def fire_scatter(c_slot, b):
    pltpu.async_copy(buf.at[b], accum.at[dstring.at[c_slot]], sem_b.at[b],
                     add=True)

  def wait_scatter(b):
    pltpu.make_async_copy(buf.at[b], accum.at[dstring.at[0]],
                          sem_b.at[b]).wait()

  def process(c_slot, b):
    wait_gather(b)
    groups = [(off, 0, L) for off in range(0, K - L + 1, L)]
    if K % L:
      groups.append((K - L, L - K % L, L))
    for off, lo, hi in groups:
      wvec = wring[b, pl.ds(off, L)]
      for jj in range(lo, hi):
        j = off + jj
        wb = wvec[jj]
        for t in range(D // L):
          sl = pl.ds(t * L, L)
          buf[b, j, sl] = buf[b, j, sl] * wb
    fire_scatter(c_slot, b)

  for s in range(NESLOT):
    fire_edge(s, s)
  for b in range(NBUF):
    wait_edge(b)
    fire_gather(b, b, b)

  def outer(g, carry):
    for b in range(NBUF):
      c = g * NBUF + b
      process(lax.rem(c, NESLOT), b)
    for b in range(NBUF):
      c = g * NBUF + b
      wait_scatter(b)

      @pl.when(c + NESLOT < NCHUNK)
      def _():
        fire_edge(c + NESLOT, lax.rem(c, NESLOT))
      cn = c + NBUF

      @pl.when(cn < NCHUNK)
      def _():
        nslot = lax.rem(cn, NESLOT)
        wait_edge(nslot)
        fire_gather(cn, nslot, b)
    return carry

  lax.fori_loop(0, NOUTER, outer, 0)
  for i in range(TAIL):
    process((NOUTER * NBUF + i) % NESLOT, i)
  for i in range(TAIL):
    wait_scatter(i)

  plsc.subcore_barrier()

  @pl.when(sid < WB_TILES)
  def _():
    pltpu.sync_copy(accum.at[pl.ds(sid * WB_ROWS, WB_ROWS)],
                    out.at[cid, pl.ds(sid * WB_ROWS, WB_ROWS)])


_spmm = functools.partial(
    pl.kernel,
    out_type=jax.ShapeDtypeStruct((NC, N, D), jnp.float32),
    mesh=plsc.VectorSubcoreMesh(core_axis_name="c", subcore_axis_name="s",
                                num_cores=NC, num_subcores=NS),
    compiler_params=pltpu.CompilerParams(needs_layout_passes=False),
    scratch_types=[
        pltpu.VMEM((NESLOT, K), jnp.int32),
        pltpu.VMEM((NESLOT, K), jnp.int32),
        pltpu.VMEM((NBUF, K), jnp.float32),
        pltpu.VMEM((NBUF, K, D), jnp.float32),
        pltpu.VMEM_SHARED((N, D), jnp.float32),
        pltpu.SemaphoreType.DMA((NESLOT,)),
        pltpu.SemaphoreType.DMA((NBUF,)),
    ],
)(_spmm_body)


def _mm0_body(x_ref, w_ref, o_ref):
  o_ref[...] = jnp.dot(x_ref[...], w_ref[...],
                       preferred_element_type=jnp.float32)


def _mid_body(p_ref, b_ref, w_ref, o_ref):
  h = jnp.maximum(p_ref[0] + p_ref[1] + b_ref[...], 0.0)
  o_ref[...] = jnp.dot(h, w_ref[...], preferred_element_type=jnp.float32)


def _fin_body(p_ref, b_ref, w1_ref, b1_ref, w2_ref, b2_ref, o1_ref, o2_ref):
  h = p_ref[0] + p_ref[1] + b_ref[...]
  o1_ref[...] = jnp.dot(h, w1_ref[...],
                        preferred_element_type=jnp.float32) + b1_ref[...]
  o2_ref[...] = jnp.dot(h, w2_ref[...],
                        preferred_element_type=jnp.float32) + b2_ref[...]


def _mm0(x, w):
  return pl.pallas_call(
      _mm0_body,
      grid=(GRID,),
      in_specs=[
          pl.BlockSpec((BM, D), lambda i: (i, 0)),
          pl.BlockSpec((D, D), lambda i: (0, 0)),
      ],
      out_specs=pl.BlockSpec((BM, D), lambda i: (i, 0)),
      out_shape=jax.ShapeDtypeStruct((N, D), jnp.float32),
  )(x, w)


def _mid(part, b, w):
  return pl.pallas_call(
      _mid_body,
      grid=(GRID,),
      in_specs=[
          pl.BlockSpec((NC, BM, D), lambda i: (0, i, 0)),
          pl.BlockSpec((1, D), lambda i: (0, 0)),
          pl.BlockSpec((D, D), lambda i: (0, 0)),
      ],
      out_specs=pl.BlockSpec((BM, D), lambda i: (i, 0)),
      out_shape=jax.ShapeDtypeStruct((N, D), jnp.float32),
  )(part, b.reshape(1, D), w)


def _fin(part, b, w1, b1, w2, b2):
  return pl.pallas_call(
      _fin_body,
      grid=(GRID,),
      in_specs=[
          pl.BlockSpec((NC, BM, D), lambda i: (0, i, 0)),
          pl.BlockSpec((1, D), lambda i: (0, 0)),
          pl.BlockSpec((D, D_OUT), lambda i: (0, 0)),
          pl.BlockSpec((1, D_OUT), lambda i: (0, 0)),
          pl.BlockSpec((D, N_CLU), lambda i: (0, 0)),
          pl.BlockSpec((1, N_CLU), lambda i: (0, 0)),
      ],
      out_specs=[
          pl.BlockSpec((BM, D_OUT), lambda i: (i, 0)),
          pl.BlockSpec((BM, N_CLU), lambda i: (i, 0)),
      ],
      out_shape=[
          jax.ShapeDtypeStruct((N, D_OUT), jnp.float32),
          jax.ShapeDtypeStruct((N, N_CLU), jnp.float32),
      ],
  )(part, b.reshape(1, D), w1, b1.reshape(1, D_OUT), w2, b2.reshape(1, N_CLU))


def kernel(x, edge_index, edge_weight, W0, b0, W1, b1, W2, b2, W3, b3, W4, b4,
           fc1_w, fc1_b, fc2_w, fc2_b):
  srcs = edge_index[0].reshape(NW, NCHUNK, K)
  dsts = edge_index[1].reshape(NW, NCHUNK, K)
  ws = edge_weight.reshape(NW, NCHUNK, K)
  zeros = jnp.zeros((N, D), jnp.float32)

  hw = _mm0(x, W0)
  Wn = [W1, W2, W3, W4]
  bn = [b0, b1, b2, b3]
  for i in range(4):
    part = _spmm(hw, srcs, dsts, ws, zeros)
    hw = _mid(part, bn[i], Wn[i])
  part = _spmm(hw, srcs, dsts, ws, zeros)
  return _fin(part, b4, fc1_w, fc1_b, fc2_w, fc2_b)


# speedup vs baseline: 1.5011x; 1.5011x over previous
import functools

import jax
import jax.numpy as jnp
from jax import lax
from jax.experimental import pallas as pl
from jax.experimental.pallas import tpu as pltpu
from jax.experimental.pallas import tpu_sc as plsc

N = 10000
E = 320000
D = 128
D_OUT = 64
N_CLU = 16

NC = 2
NS = 16
L = 16
NW = NC * NS
EPT = E // NW
K = 16
NCHUNK = EPT // K
NBUF = 8
NESLOT = 2 * NBUF
NOUTER = NCHUNK // NBUF
TAIL = NCHUNK - NOUTER * NBUF

WB_TILES = 10
WB_ROWS = N // WB_TILES

BM = 2000
GRID = N // BM


def _spmm_body(tab, srcs, dsts, ws, zeros, out, srcring, dstring, wring,
               buf, accum, sem_e, sem_b):
  cid = lax.axis_index("c")
  sid = lax.axis_index("s")
  wid = sid * NC + cid

  @pl.when(sid < WB_TILES)
  def _():
    pltpu.sync_copy(zeros.at[pl.ds(sid * WB_ROWS, WB_ROWS)],
                    accum.at[pl.ds(sid * WB_ROWS, WB_ROWS)])
  plsc.subcore_barrier()

  def fire_edge(c, s):
    pltpu.async_copy(srcs.at[wid, c], srcring.at[s], sem_e.at[s])
    pltpu.async_copy(dsts.at[wid, c], dstring.at[s], sem_e.at[s])

  def wait_edge(s):
    pltpu.make_async_copy(srcs.at[wid, 0], srcring.at[s], sem_e.at[s]).wait()
    pltpu.make_async_copy(dsts.at[wid, 0], dstring.at[s], sem_e.at[s]).wait()

  def fire_gather(c, c_slot, b):
    pltpu.async_copy(tab.at[srcring.at[c_slot]], buf.at[b], sem_b.at[b])
    pltpu.async_copy(ws.at[wid, c], wring.at[b], sem_b.at[b])

  def wait_gather(b):
    pltpu.make_async_copy(tab.at[srcring.at[0]], buf.at[b],
                          sem_b.at[b]).wait()
    pltpu.make_async_copy(ws.at[wid, 0], wring.at[b], sem_b.at[b]).wait()

  def fire_scatter(c_slot, b):
    pltpu.async_copy(buf.at[b], accum.at[dstring.at[c_slot]], sem_b.at[b],
                     add=True)

  def wait_scatter(b):
    pltpu.make_async_copy(buf.at[b], accum.at[dstring.at[0]],
                          sem_b.at[b]).wait()

  def process(c_slot, b):
    wait_gather(b)
    groups = [(off, 0, L) for off in range(0, K - L + 1, L)]
    if K % L:
      groups.append((K - L, L - K % L, L))
    for off, lo, hi in groups:
      wvec = wring[b, pl.ds(off, L)]
      for jj in range(lo, hi):
        j = off + jj
        wb = wvec[jj]
        for t in range(D // L):
          sl = pl.ds(t * L, L)
          buf[b, j, sl] = buf[b, j, sl] * wb
    fire_scatter(c_slot, b)

  for s in range(NESLOT):
    fire_edge(s, s)
  for b in range(NBUF):
    wait_edge(b)
    fire_gather(b, b, b)

  def outer(g, carry):
    for b in range(NBUF):
      c = g * NBUF + b
      process(lax.rem(c, NESLOT), b)
    for b in range(NBUF):
      c = g * NBUF + b
      wait_scatter(b)

      @pl.when(c + NESLOT < NCHUNK)
      def _():
        fire_edge(c + NESLOT, lax.rem(c, NESLOT))
      cn = c + NBUF

      @pl.when(cn < NCHUNK)
      def _():
        nslot = lax.rem(cn, NESLOT)
        wait_edge(nslot)
        fire_gather(cn, nslot, b)
    return carry

  lax.fori_loop(0, NOUTER, outer, 0)
  for i in range(TAIL):
    process((NOUTER * NBUF + i) % NESLOT, i)
  for i in range(TAIL):
    wait_scatter(i)

  plsc.subcore_barrier()

  @pl.when(sid < WB_TILES)
  def _():
    pltpu.sync_copy(accum.at[pl.ds(sid * WB_ROWS, WB_ROWS)],
                    out.at[cid, pl.ds(sid * WB_ROWS, WB_ROWS)])


_spmm = functools.partial(
    pl.kernel,
    out_type=jax.ShapeDtypeStruct((NC, N, D), jnp.float32),
    mesh=plsc.VectorSubcoreMesh(core_axis_name="c", subcore_axis_name="s",
                                num_cores=NC, num_subcores=NS),
    compiler_params=pltpu.CompilerParams(needs_layout_passes=False),
    scratch_types=[
        pltpu.VMEM((NESLOT, K), jnp.int32),
        pltpu.VMEM((NESLOT, K), jnp.int32),
        pltpu.VMEM((NBUF, K), jnp.float32),
        pltpu.VMEM((NBUF, K, D), jnp.float32),
        pltpu.VMEM_SHARED((N, D), jnp.float32),
        pltpu.SemaphoreType.DMA((NESLOT,)),
        pltpu.SemaphoreType.DMA((NBUF,)),
    ],
)(_spmm_body)


def _mm0_body(x_ref, w_ref, o_ref):
  o_ref[...] = jnp.dot(x_ref[...], w_ref[...],
                       preferred_element_type=jnp.float32)


def _mid_body(p_ref, b_ref, w_ref, o_ref):
  h = jnp.maximum(p_ref[0] + p_ref[1] + b_ref[...], 0.0)
  o_ref[...] = jnp.dot(h, w_ref[...], preferred_element_type=jnp.float32)


def _fin_body(p_ref, b_ref, w1_ref, b1_ref, w2_ref, b2_ref, o1_ref, o2_ref):
  h = p_ref[0] + p_ref[1] + b_ref[...]
  o1_ref[...] = jnp.dot(h, w1_ref[...],
                        preferred_element_type=jnp.float32) + b1_ref[...]
  o2_ref[...] = jnp.dot(h, w2_ref[...],
                        preferred_element_type=jnp.float32) + b2_ref[...]


def _mm0(x, w):
  return pl.pallas_call(
      _mm0_body,
      grid=(GRID,),
      in_specs=[
          pl.BlockSpec((BM, D), lambda i: (i, 0)),
          pl.BlockSpec((D, D), lambda i: (0, 0)),
      ],
      out_specs=pl.BlockSpec((BM, D), lambda i: (i, 0)),
      out_shape=jax.ShapeDtypeStruct((N, D), jnp.float32),
  )(x, w)


def _mid(part, b, w):
  return pl.pallas_call(
      _mid_body,
      grid=(GRID,),
      in_specs=[
          pl.BlockSpec((NC, BM, D), lambda i: (0, i, 0)),
          pl.BlockSpec((1, D), lambda i: (0, 0)),
          pl.BlockSpec((D, D), lambda i: (0, 0)),
      ],
      out_specs=pl.BlockSpec((BM, D), lambda i: (i, 0)),
      out_shape=jax.ShapeDtypeStruct((N, D), jnp.float32),
  )(part, b.reshape(1, D), w)


def _fin(part, b, w1, b1, w2, b2):
  return pl.pallas_call(
      _fin_body,
      grid=(GRID,),
      in_specs=[
          pl.BlockSpec((NC, BM, D), lambda i: (0, i, 0)),
          pl.BlockSpec((1, D), lambda i: (0, 0)),
          pl.BlockSpec((D, D_OUT), lambda i: (0, 0)),
          pl.BlockSpec((1, D_OUT), lambda i: (0, 0)),
          pl.BlockSpec((D, N_CLU), lambda i: (0, 0)),
          pl.BlockSpec((1, N_CLU), lambda i: (0, 0)),
      ],
      out_specs=[
          pl.BlockSpec((BM, D_OUT), lambda i: (i, 0)),
          pl.BlockSpec((BM, N_CLU), lambda i: (i, 0)),
      ],
      out_shape=[
          jax.ShapeDtypeStruct((N, D_OUT), jnp.float32),
          jax.ShapeDtypeStruct((N, N_CLU), jnp.float32),
      ],
  )(part, b.reshape(1, D), w1, b1.reshape(1, D_OUT), w2, b2.reshape(1, N_CLU))


def kernel(x, edge_index, edge_weight, W0, b0, W1, b1, W2, b2, W3, b3, W4, b4,
           fc1_w, fc1_b, fc2_w, fc2_b):
  srcs = edge_index[0].reshape(NW, NCHUNK, K)
  dsts = edge_index[1].reshape(NW, NCHUNK, K)
  ws = edge_weight.reshape(NW, NCHUNK, K)
  zeros = jnp.zeros((N, D), jnp.float32)

  hw = _mm0(x, W0)
  Wn = [W1, W2, W3, W4]
  bn = [b0, b1, b2, b3]
  for i in range(4):
    part = _spmm(hw, srcs, dsts, ws, zeros)
    hw = _mid(part, bn[i], Wn[i])
  part = _spmm(hw, srcs, dsts, ws, zeros)
  return _fin(part, b4, fc1_w, fc1_b, fc2_w, fc2_b)
